# Initial kernel scaffold; baseline (speedup 1.0000x reference)
import math

import jax
import jax.numpy as jnp
from jax.experimental import pallas as pl

_HID = 64
_NL = 2
_ALPHA = 0.1
_LAMDA = 1.0
_EPS = 1e-5
_F32 = jnp.float32
_PREC = jax.lax.Precision.HIGHEST


def _proj_kernel(x_ref, w_ref, b_ref, h_ref):
    x = x_ref[0]
    h = jax.lax.dot_general(x, w_ref[...], (((1,), (1,)), ((), ())),
                            precision=_PREC, preferred_element_type=_F32)
    h_ref[0] = h + b_ref[...]


def _deg_kernel(adj_ref, dinv_ref):
    a = adj_ref[0].astype(_F32)
    ones = jnp.ones((a.shape[0], 1), _F32)
    deg = jax.lax.dot_general(a, ones, (((0,), (0,)), ((), ())),
                              precision=_PREC, preferred_element_type=_F32)
    dinv_ref[0] = jax.lax.rsqrt(deg + 1.0)


def _make_layer_kernel(beta, bd):
    def _layer(h_ref, adj_ref, dinv_ref, g_ref, b_ref, w_ref, hout_ref):
        h = h_ref[0]
        mu = jnp.mean(h, axis=1, keepdims=True)
        xc = h - mu
        var = jnp.mean(xc * xc, axis=1, keepdims=True)
        t = xc * jax.lax.rsqrt(var + _EPS) * g_ref[...] + b_ref[...]
        t = jnp.maximum(t, 0.0)
        dinv = dinv_ref[0]
        gm = dinv * t
        a = adj_ref[0].astype(_F32)
        agg = jax.lax.dot_general(a, gm, (((0,), (0,)), ((), ())),
                                  precision=_PREC, preferred_element_type=_F32)
        row = pl.program_id(1) * bd
        t_d = jax.lax.dynamic_slice(t, (row, 0), (bd, _HID))
        gm_d = jax.lax.dynamic_slice(gm, (row, 0), (bd, _HID))
        dinv_d = jax.lax.dynamic_slice(dinv, (row, 0), (bd, 1))
        h_d = jax.lax.dynamic_slice(h, (row, 0), (bd, _HID))
        xx = (1.0 - _ALPHA) * (dinv_d * (agg + gm_d)) + _ALPHA * t_d
        conv = (1.0 - beta) * xx + beta * jax.lax.dot_general(
            xx, w_ref[...], (((1,), (0,)), ((), ())),
            precision=_PREC, preferred_element_type=_F32)
        hout_ref[0] = h_d + conv
    return _layer


def _pred_kernel(h_ref, w_ref, b_ref, o_ref):
    o = jax.lax.dot_general(h_ref[0], w_ref[...], (((1,), (1,)), ((), ())),
                            precision=_PREC, preferred_element_type=_F32)
    o_ref[0] = o + b_ref[...]


def kernel(x, adj, proj_W, proj_b, ln_g, ln_b, conv_W, pred_W, pred_b):
    bs, N, D = x.shape
    BC = 512
    BD = 256

    h = pl.pallas_call(
        _proj_kernel,
        grid=(bs,),
        in_specs=[
            pl.BlockSpec((1, N, D), lambda b: (b, 0, 0)),
            pl.BlockSpec((_HID, D), lambda b: (0, 0)),
            pl.BlockSpec((1, _HID), lambda b: (0, 0)),
        ],
        out_specs=pl.BlockSpec((1, N, _HID), lambda b: (b, 0, 0)),
        out_shape=jax.ShapeDtypeStruct((bs, N, _HID), _F32),
    )(x, proj_W, proj_b.reshape(1, _HID))

    dinv = pl.pallas_call(
        _deg_kernel,
        grid=(bs, N // BC),
        in_specs=[pl.BlockSpec((1, N, BC), lambda b, c: (b, 0, c))],
        out_specs=pl.BlockSpec((1, BC, 1), lambda b, c: (b, c, 0)),
        out_shape=jax.ShapeDtypeStruct((bs, N, 1), _F32),
    )(adj)

    for l in range(_NL):
        beta = math.log(_LAMDA / (l + 1) + 1.0)
        h = pl.pallas_call(
            _make_layer_kernel(beta, BD),
            grid=(bs, N // BD),
            in_specs=[
                pl.BlockSpec((1, N, _HID), lambda b, d: (b, 0, 0)),
                pl.BlockSpec((1, N, BD), lambda b, d: (b, 0, d)),
                pl.BlockSpec((1, N, 1), lambda b, d: (b, 0, 0)),
                pl.BlockSpec((1, _HID), lambda b, d: (0, 0)),
                pl.BlockSpec((1, _HID), lambda b, d: (0, 0)),
                pl.BlockSpec((_HID, _HID), lambda b, d: (0, 0)),
            ],
            out_specs=pl.BlockSpec((1, BD, _HID), lambda b, d: (b, d, 0)),
            out_shape=jax.ShapeDtypeStruct((bs, N, _HID), _F32),
        )(h, adj, dinv, ln_g[l:l + 1], ln_b[l:l + 1], conv_W[l])

    out = pl.pallas_call(
        _pred_kernel,
        grid=(bs,),
        in_specs=[
            pl.BlockSpec((1, N, _HID), lambda b: (b, 0, 0)),
            pl.BlockSpec((1, _HID), lambda b: (0, 0)),
            pl.BlockSpec((1, 1), lambda b: (0, 0)),
        ],
        out_specs=pl.BlockSpec((1, N, 1), lambda b: (b, 0, 0)),
        out_shape=jax.ShapeDtypeStruct((bs, N, 1), _F32),
    )(h, pred_W, pred_b.reshape(1, 1))
    return out


# speedup vs baseline: 1894.8162x; 1894.8162x over previous
import math

import jax
import jax.numpy as jnp
from jax.experimental import pallas as pl

_HID = 64
_NL = 2
_ALPHA = 0.1
_LAMDA = 1.0
_EPS = 1e-5
_F32 = jnp.float32
_PREC = jax.lax.Precision.HIGHEST


def _proj_kernel(x_ref, w_ref, b_ref, h_ref):
    x = x_ref[0]
    h = jax.lax.dot_general(x, w_ref[...], (((1,), (1,)), ((), ())),
                            precision=_PREC, preferred_element_type=_F32)
    h_ref[0] = h + b_ref[...]


def _deg_kernel(adj_ref, dinv_ref):
    a = adj_ref[0].astype(_F32)
    ones = jnp.ones((a.shape[0], 1), _F32)
    deg = jax.lax.dot_general(a, ones, (((0,), (0,)), ((), ())),
                              precision=_PREC, preferred_element_type=_F32)
    dinv_ref[0] = jax.lax.rsqrt(deg + 1.0)


def _ln_relu(h, g, b):
    mu = jnp.mean(h, axis=1, keepdims=True)
    xc = h - mu
    var = jnp.mean(xc * xc, axis=1, keepdims=True)
    t = xc * jax.lax.rsqrt(var + _EPS) * g + b
    return jnp.maximum(t, 0.0)


def _make_layer_kernel(beta, bd):
    def _layer(h_ref, adj_ref, dinv_ref, g_ref, b_ref, w_ref, hout_ref):
        h = h_ref[0]
        t = _ln_relu(h, g_ref[...], b_ref[...])
        dinv = dinv_ref[0]
        gm = dinv * t
        a = adj_ref[0].astype(_F32)
        agg = jax.lax.dot_general(a, gm, (((0,), (0,)), ((), ())),
                                  precision=_PREC, preferred_element_type=_F32)
        row = pl.program_id(1) * bd
        h_d = h_ref[0, pl.ds(row, bd), :]
        dinv_d = dinv_ref[0, pl.ds(row, bd), :]
        t_d = _ln_relu(h_d, g_ref[...], b_ref[...])
        gm_d = dinv_d * t_d
        xx = (1.0 - _ALPHA) * (dinv_d * (agg + gm_d)) + _ALPHA * t_d
        conv = (1.0 - beta) * xx + beta * jax.lax.dot_general(
            xx, w_ref[...], (((1,), (0,)), ((), ())),
            precision=_PREC, preferred_element_type=_F32)
        hout_ref[0] = h_d + conv
    return _layer


def _pred_kernel(h_ref, w_ref, b_ref, o_ref):
    o = jnp.sum(h_ref[0] * w_ref[...], axis=1, keepdims=True)
    o_ref[0] = o + b_ref[0, 0]


def kernel(x, adj, proj_W, proj_b, ln_g, ln_b, conv_W, pred_W, pred_b):
    bs, N, D = x.shape
    BC = 512
    BD = 256

    h = pl.pallas_call(
        _proj_kernel,
        grid=(bs,),
        in_specs=[
            pl.BlockSpec((1, N, D), lambda b: (b, 0, 0)),
            pl.BlockSpec((_HID, D), lambda b: (0, 0)),
            pl.BlockSpec((1, _HID), lambda b: (0, 0)),
        ],
        out_specs=pl.BlockSpec((1, N, _HID), lambda b: (b, 0, 0)),
        out_shape=jax.ShapeDtypeStruct((bs, N, _HID), _F32),
    )(x, proj_W, proj_b.reshape(1, _HID))

    dinv = pl.pallas_call(
        _deg_kernel,
        grid=(bs, N // BC),
        in_specs=[pl.BlockSpec((1, N, BC), lambda b, c: (b, 0, c))],
        out_specs=pl.BlockSpec((1, BC, 1), lambda b, c: (b, c, 0)),
        out_shape=jax.ShapeDtypeStruct((bs, N, 1), _F32),
    )(adj)

    for l in range(_NL):
        beta = math.log(_LAMDA / (l + 1) + 1.0)
        h = pl.pallas_call(
            _make_layer_kernel(beta, BD),
            grid=(bs, N // BD),
            in_specs=[
                pl.BlockSpec((1, N, _HID), lambda b, d: (b, 0, 0)),
                pl.BlockSpec((1, N, BD), lambda b, d: (b, 0, d)),
                pl.BlockSpec((1, N, 1), lambda b, d: (b, 0, 0)),
                pl.BlockSpec((1, _HID), lambda b, d: (0, 0)),
                pl.BlockSpec((1, _HID), lambda b, d: (0, 0)),
                pl.BlockSpec((_HID, _HID), lambda b, d: (0, 0)),
            ],
            out_specs=pl.BlockSpec((1, BD, _HID), lambda b, d: (b, d, 0)),
            out_shape=jax.ShapeDtypeStruct((bs, N, _HID), _F32),
        )(h, adj, dinv, ln_g[l:l + 1], ln_b[l:l + 1], conv_W[l])

    out = pl.pallas_call(
        _pred_kernel,
        grid=(bs,),
        in_specs=[
            pl.BlockSpec((1, N, _HID), lambda b: (b, 0, 0)),
            pl.BlockSpec((1, _HID), lambda b: (0, 0)),
            pl.BlockSpec((1, 1), lambda b: (0, 0)),
        ],
        out_specs=pl.BlockSpec((1, N, 1), lambda b: (b, 0, 0)),
        out_shape=jax.ShapeDtypeStruct((bs, N, 1), _F32),
    )(h, pred_W, pred_b.reshape(1, 1))
    return out


# speedup vs baseline: 1901.1864x; 1.0034x over previous
import math

import jax
import jax.numpy as jnp
from jax.experimental import pallas as pl

_HID = 64
_NL = 2
_ALPHA = 0.1
_LAMDA = 1.0
_EPS = 1e-5
_F32 = jnp.float32
_PREC = jax.lax.Precision.HIGHEST


def _proj_kernel(x_ref, w_ref, b_ref, h_ref):
    x = x_ref[0]
    h = jax.lax.dot_general(x, w_ref[...], (((1,), (1,)), ((), ())),
                            precision=_PREC, preferred_element_type=_F32)
    h_ref[0] = h + b_ref[...]


def _deg_kernel(adj_ref, dinv_ref, adj8_ref):
    a = adj_ref[0].astype(_F32)
    ones = jnp.ones((a.shape[0], 1), _F32)
    deg = jax.lax.dot_general(a, ones, (((0,), (0,)), ((), ())),
                              precision=_PREC, preferred_element_type=_F32)
    dinv_ref[0] = jax.lax.rsqrt(deg + 1.0)
    adj8_ref[0] = adj_ref[0].astype(jnp.int8)


def _ln_relu(h, g, b):
    mu = jnp.mean(h, axis=1, keepdims=True)
    xc = h - mu
    var = jnp.mean(xc * xc, axis=1, keepdims=True)
    t = xc * jax.lax.rsqrt(var + _EPS) * g + b
    return jnp.maximum(t, 0.0)


def _make_layer_kernel(beta, bd):
    def _layer(h_ref, adj_ref, dinv_ref, g_ref, b_ref, w_ref, hout_ref):
        h = h_ref[0]
        t = _ln_relu(h, g_ref[...], b_ref[...])
        dinv = dinv_ref[0]
        gm = dinv * t
        a = adj_ref[0].astype(_F32)
        agg = jax.lax.dot_general(a, gm, (((0,), (0,)), ((), ())),
                                  precision=_PREC, preferred_element_type=_F32)
        row = pl.program_id(1) * bd
        h_d = h_ref[0, pl.ds(row, bd), :]
        dinv_d = dinv_ref[0, pl.ds(row, bd), :]
        t_d = _ln_relu(h_d, g_ref[...], b_ref[...])
        gm_d = dinv_d * t_d
        xx = (1.0 - _ALPHA) * (dinv_d * (agg + gm_d)) + _ALPHA * t_d
        conv = (1.0 - beta) * xx + beta * jax.lax.dot_general(
            xx, w_ref[...], (((1,), (0,)), ((), ())),
            precision=_PREC, preferred_element_type=_F32)
        hout_ref[0] = h_d + conv
    return _layer


def _pred_kernel(h_ref, w_ref, b_ref, o_ref):
    o = jnp.sum(h_ref[0] * w_ref[...], axis=1, keepdims=True)
    o_ref[0] = o + b_ref[0, 0]


def kernel(x, adj, proj_W, proj_b, ln_g, ln_b, conv_W, pred_W, pred_b):
    bs, N, D = x.shape
    BC = 512
    BD = 256

    h = pl.pallas_call(
        _proj_kernel,
        grid=(bs,),
        in_specs=[
            pl.BlockSpec((1, N, D), lambda b: (b, 0, 0)),
            pl.BlockSpec((_HID, D), lambda b: (0, 0)),
            pl.BlockSpec((1, _HID), lambda b: (0, 0)),
        ],
        out_specs=pl.BlockSpec((1, N, _HID), lambda b: (b, 0, 0)),
        out_shape=jax.ShapeDtypeStruct((bs, N, _HID), _F32),
    )(x, proj_W, proj_b.reshape(1, _HID))

    dinv, adj8 = pl.pallas_call(
        _deg_kernel,
        grid=(bs, N // BC),
        in_specs=[pl.BlockSpec((1, N, BC), lambda b, c: (b, 0, c))],
        out_specs=[
            pl.BlockSpec((1, BC, 1), lambda b, c: (b, c, 0)),
            pl.BlockSpec((1, N, BC), lambda b, c: (b, 0, c)),
        ],
        out_shape=[
            jax.ShapeDtypeStruct((bs, N, 1), _F32),
            jax.ShapeDtypeStruct((bs, N, N), jnp.int8),
        ],
    )(adj)

    for l in range(_NL):
        beta = math.log(_LAMDA / (l + 1) + 1.0)
        h = pl.pallas_call(
            _make_layer_kernel(beta, BD),
            grid=(bs, N // BD),
            in_specs=[
                pl.BlockSpec((1, N, _HID), lambda b, d: (b, 0, 0)),
                pl.BlockSpec((1, N, BD), lambda b, d: (b, 0, d)),
                pl.BlockSpec((1, N, 1), lambda b, d: (b, 0, 0)),
                pl.BlockSpec((1, _HID), lambda b, d: (0, 0)),
                pl.BlockSpec((1, _HID), lambda b, d: (0, 0)),
                pl.BlockSpec((_HID, _HID), lambda b, d: (0, 0)),
            ],
            out_specs=pl.BlockSpec((1, BD, _HID), lambda b, d: (b, d, 0)),
            out_shape=jax.ShapeDtypeStruct((bs, N, _HID), _F32),
        )(h, adj8, dinv, ln_g[l:l + 1], ln_b[l:l + 1], conv_W[l])

    out = pl.pallas_call(
        _pred_kernel,
        grid=(bs,),
        in_specs=[
            pl.BlockSpec((1, N, _HID), lambda b: (b, 0, 0)),
            pl.BlockSpec((1, _HID), lambda b: (0, 0)),
            pl.BlockSpec((1, 1), lambda b: (0, 0)),
        ],
        out_specs=pl.BlockSpec((1, N, 1), lambda b: (b, 0, 0)),
        out_shape=jax.ShapeDtypeStruct((bs, N, 1), _F32),
    )(h, pred_W, pred_b.reshape(1, 1))
    return out


# speedup vs baseline: 4023.1719x; 2.1161x over previous
import math

import jax
import jax.numpy as jnp
from jax.experimental import pallas as pl

_HID = 64
_NL = 2
_ALPHA = 0.1
_LAMDA = 1.0
_EPS = 1e-5
_F32 = jnp.float32
_BF16 = jnp.bfloat16
_PREC = jax.lax.Precision.HIGHEST


def _proj_kernel(x_ref, w_ref, b_ref, h_ref):
    x = x_ref[0]
    h = jax.lax.dot_general(x, w_ref[...], (((1,), (1,)), ((), ())),
                            precision=_PREC, preferred_element_type=_F32)
    h_ref[0] = h + b_ref[...]


def _deg_kernel(adj_ref, dinv_ref, adj16_ref):
    a = adj_ref[0].astype(_BF16)
    ones = jnp.ones((a.shape[0], 1), _BF16)
    deg = jax.lax.dot_general(a, ones, (((0,), (0,)), ((), ())),
                              preferred_element_type=_F32)
    dinv_ref[0] = jax.lax.rsqrt(deg + 1.0)
    adj16_ref[0] = a


def _ln_relu(h, g, b):
    mu = jnp.mean(h, axis=1, keepdims=True)
    xc = h - mu
    var = jnp.mean(xc * xc, axis=1, keepdims=True)
    t = xc * jax.lax.rsqrt(var + _EPS) * g + b
    return jnp.maximum(t, 0.0)


def _make_layer_kernel(beta):
    def _layer(h_ref, adj_ref, dinv_ref, g_ref, b_ref, w_ref, hout_ref):
        h = h_ref[0]
        t = _ln_relu(h, g_ref[...], b_ref[...])
        dinv = dinv_ref[0]
        gm = dinv * t
        a = adj_ref[0]
        agg = jax.lax.dot_general(a, gm.astype(_BF16), (((0,), (0,)), ((), ())),
                                  preferred_element_type=_F32)
        xx = (1.0 - _ALPHA) * (dinv * (agg + gm)) + _ALPHA * t
        conv = (1.0 - beta) * xx + beta * jax.lax.dot_general(
            xx, w_ref[...], (((1,), (0,)), ((), ())),
            precision=_PREC, preferred_element_type=_F32)
        hout_ref[0] = h + conv
    return _layer


def _pred_kernel(h_ref, w_ref, b_ref, o_ref):
    o = jnp.sum(h_ref[0] * w_ref[...], axis=1, keepdims=True)
    o_ref[0] = o + b_ref[0, 0]


def kernel(x, adj, proj_W, proj_b, ln_g, ln_b, conv_W, pred_W, pred_b):
    bs, N, D = x.shape
    BC = 512

    h = pl.pallas_call(
        _proj_kernel,
        grid=(bs,),
        in_specs=[
            pl.BlockSpec((1, N, D), lambda b: (b, 0, 0)),
            pl.BlockSpec((_HID, D), lambda b: (0, 0)),
            pl.BlockSpec((1, _HID), lambda b: (0, 0)),
        ],
        out_specs=pl.BlockSpec((1, N, _HID), lambda b: (b, 0, 0)),
        out_shape=jax.ShapeDtypeStruct((bs, N, _HID), _F32),
    )(x, proj_W, proj_b.reshape(1, _HID))

    dinv, adj16 = pl.pallas_call(
        _deg_kernel,
        grid=(bs, N // BC),
        in_specs=[pl.BlockSpec((1, N, BC), lambda b, c: (b, 0, c))],
        out_specs=[
            pl.BlockSpec((1, BC, 1), lambda b, c: (b, c, 0)),
            pl.BlockSpec((1, N, BC), lambda b, c: (b, 0, c)),
        ],
        out_shape=[
            jax.ShapeDtypeStruct((bs, N, 1), _F32),
            jax.ShapeDtypeStruct((bs, N, N), _BF16),
        ],
    )(adj)

    for l in range(_NL):
        beta = math.log(_LAMDA / (l + 1) + 1.0)
        h = pl.pallas_call(
            _make_layer_kernel(beta),
            grid=(bs,),
            in_specs=[
                pl.BlockSpec((1, N, _HID), lambda b: (b, 0, 0)),
                pl.BlockSpec((1, N, N), lambda b: (b, 0, 0)),
                pl.BlockSpec((1, N, 1), lambda b: (b, 0, 0)),
                pl.BlockSpec((1, _HID), lambda b: (0, 0)),
                pl.BlockSpec((1, _HID), lambda b: (0, 0)),
                pl.BlockSpec((_HID, _HID), lambda b: (0, 0)),
            ],
            out_specs=pl.BlockSpec((1, N, _HID), lambda b: (b, 0, 0)),
            out_shape=jax.ShapeDtypeStruct((bs, N, _HID), _F32),
        )(h, adj16, dinv, ln_g[l:l + 1], ln_b[l:l + 1], conv_W[l])

    out = pl.pallas_call(
        _pred_kernel,
        grid=(bs,),
        in_specs=[
            pl.BlockSpec((1, N, _HID), lambda b: (b, 0, 0)),
            pl.BlockSpec((1, _HID), lambda b: (0, 0)),
            pl.BlockSpec((1, 1), lambda b: (0, 0)),
        ],
        out_specs=pl.BlockSpec((1, N, 1), lambda b: (b, 0, 0)),
        out_shape=jax.ShapeDtypeStruct((bs, N, 1), _F32),
    )(h, pred_W, pred_b.reshape(1, 1))
    return out


# speedup vs baseline: 5334.8761x; 1.3260x over previous
import math

import jax
import jax.numpy as jnp
from jax.experimental import pallas as pl

_HID = 64
_NL = 2
_ALPHA = 0.1
_LAMDA = 1.0
_EPS = 1e-5
_F32 = jnp.float32
_BF16 = jnp.bfloat16
_PREC = jax.lax.Precision.HIGHEST


def _proj_kernel(x_ref, w_ref, b_ref, h_ref):
    x = x_ref[0]
    h = jax.lax.dot_general(x, w_ref[...], (((1,), (1,)), ((), ())),
                            precision=_PREC, preferred_element_type=_F32)
    h_ref[0] = h + b_ref[...]


def _ln_relu(h, g, b):
    mu = jnp.mean(h, axis=1, keepdims=True)
    xc = h - mu
    var = jnp.mean(xc * xc, axis=1, keepdims=True)
    t = xc * jax.lax.rsqrt(var + _EPS) * g + b
    return jnp.maximum(t, 0.0)


def _gcn2(a, t, dinv, w, beta):
    gm = dinv * t
    agg = jax.lax.dot_general(a, gm.astype(_BF16), (((0,), (0,)), ((), ())),
                              preferred_element_type=_F32)
    xx = (1.0 - _ALPHA) * (dinv * (agg + gm)) + _ALPHA * t
    return (1.0 - beta) * xx + beta * jax.lax.dot_general(
        xx, w, (((1,), (0,)), ((), ())),
        precision=_PREC, preferred_element_type=_F32)


def _make_layer1_kernel(beta):
    def _k(h_ref, adj_ref, g_ref, b_ref, w_ref, hout_ref, adj8_ref, dinv_ref):
        a32 = adj_ref[0]
        a = a32.astype(_BF16)
        adj8_ref[0] = a32.astype(jnp.int8)
        ones = jnp.ones((a.shape[0], 1), _BF16)
        deg = jax.lax.dot_general(a, ones, (((0,), (0,)), ((), ())),
                                  preferred_element_type=_F32)
        dinv = jax.lax.rsqrt(deg + 1.0)
        dinv_ref[0] = dinv
        h = h_ref[0]
        t = _ln_relu(h, g_ref[...], b_ref[...])
        hout_ref[0] = h + _gcn2(a, t, dinv, w_ref[...], beta)
    return _k


def _make_layer2_pred_kernel(beta):
    def _k(h_ref, adj8_ref, dinv_ref, g_ref, b_ref, w_ref, pw_ref, pb_ref,
           o_ref):
        a = adj8_ref[0].astype(_BF16)
        h = h_ref[0]
        t = _ln_relu(h, g_ref[...], b_ref[...])
        hf = h + _gcn2(a, t, dinv_ref[0], w_ref[...], beta)
        o = jnp.sum(hf * pw_ref[...], axis=1, keepdims=True)
        o_ref[0] = o + pb_ref[0, 0]
    return _k


def kernel(x, adj, proj_W, proj_b, ln_g, ln_b, conv_W, pred_W, pred_b):
    bs, N, D = x.shape
    beta1 = math.log(_LAMDA / 1.0 + 1.0)
    beta2 = math.log(_LAMDA / 2.0 + 1.0)

    h = pl.pallas_call(
        _proj_kernel,
        grid=(bs,),
        in_specs=[
            pl.BlockSpec((1, N, D), lambda b: (b, 0, 0)),
            pl.BlockSpec((_HID, D), lambda b: (0, 0)),
            pl.BlockSpec((1, _HID), lambda b: (0, 0)),
        ],
        out_specs=pl.BlockSpec((1, N, _HID), lambda b: (b, 0, 0)),
        out_shape=jax.ShapeDtypeStruct((bs, N, _HID), _F32),
    )(x, proj_W, proj_b.reshape(1, _HID))

    h, adj8, dinv = pl.pallas_call(
        _make_layer1_kernel(beta1),
        grid=(bs,),
        in_specs=[
            pl.BlockSpec((1, N, _HID), lambda b: (b, 0, 0)),
            pl.BlockSpec((1, N, N), lambda b: (b, 0, 0)),
            pl.BlockSpec((1, _HID), lambda b: (0, 0)),
            pl.BlockSpec((1, _HID), lambda b: (0, 0)),
            pl.BlockSpec((_HID, _HID), lambda b: (0, 0)),
        ],
        out_specs=[
            pl.BlockSpec((1, N, _HID), lambda b: (b, 0, 0)),
            pl.BlockSpec((1, N, N), lambda b: (b, 0, 0)),
            pl.BlockSpec((1, N, 1), lambda b: (b, 0, 0)),
        ],
        out_shape=[
            jax.ShapeDtypeStruct((bs, N, _HID), _F32),
            jax.ShapeDtypeStruct((bs, N, N), jnp.int8),
            jax.ShapeDtypeStruct((bs, N, 1), _F32),
        ],
    )(h, adj, ln_g[0:1], ln_b[0:1], conv_W[0])

    out = pl.pallas_call(
        _make_layer2_pred_kernel(beta2),
        grid=(bs,),
        in_specs=[
            pl.BlockSpec((1, N, _HID), lambda b: (b, 0, 0)),
            pl.BlockSpec((1, N, N), lambda b: (b, 0, 0)),
            pl.BlockSpec((1, N, 1), lambda b: (b, 0, 0)),
            pl.BlockSpec((1, _HID), lambda b: (0, 0)),
            pl.BlockSpec((1, _HID), lambda b: (0, 0)),
            pl.BlockSpec((_HID, _HID), lambda b: (0, 0)),
            pl.BlockSpec((1, _HID), lambda b: (0, 0)),
            pl.BlockSpec((1, 1), lambda b: (0, 0)),
        ],
        out_specs=pl.BlockSpec((1, N, 1), lambda b: (b, 0, 0)),
        out_shape=jax.ShapeDtypeStruct((bs, N, 1), _F32),
    )(h, adj8, dinv, ln_g[1:2], ln_b[1:2], conv_W[1], pred_W,
      pred_b.reshape(1, 1))
    return out
